# baseline (speedup 1.0000x reference)
import functools

import jax
import jax.numpy as jnp
from jax import lax
from jax.experimental import pallas as pl
from jax.experimental.pallas import tpu as pltpu
from jax.experimental.pallas import tpu_sc as plsc

N = 10000
E = 320000
D_IN = 128
D_HID = 64
D_OUT = 128
BN_EPS = 1e-5

NC = 2
NS = 16
NW = NC * NS
EPW = E // NW
CHUNK = 80
NCH = EPW // CHUNK
NPAD = 10240
RPT = NPAD // NS

_mesh = plsc.VectorSubcoreMesh(
    core_axis_name="c", subcore_axis_name="s", num_cores=NC, num_subcores=NS
)
_sc_params = pltpu.CompilerParams(
    needs_layout_passes=False, use_tc_tiling_on_sc=False
)


@functools.partial(
    pl.kernel,
    out_type=jax.ShapeDtypeStruct((NW, NPAD), jnp.float32),
    mesh=_mesh,
    compiler_params=_sc_params,
    scratch_types=[
        pltpu.VMEM((EPW,), jnp.int32),
        pltpu.VMEM((EPW,), jnp.float32),
        pltpu.VMEM((NPAD,), jnp.float32),
    ],
)
def _deg_kernel(ei_hbm, w_hbm, out_hbm, col_v, w_v, acc):
    cid = lax.axis_index("c")
    sid = lax.axis_index("s")
    wid = sid * NC + cid
    pltpu.sync_copy(ei_hbm.at[1, pl.ds(wid * EPW, EPW)], col_v)
    pltpu.sync_copy(w_hbm.at[pl.ds(wid * EPW, EPW)], w_v)

    zero16 = jnp.zeros((16,), jnp.float32)

    def zero_body(i, carry):
        acc[pl.ds(i * 16, 16)] = zero16
        return carry

    lax.fori_loop(0, NPAD // 16, zero_body, 0)

    def edge_body(j, carry):
        cv = col_v[pl.ds(j * 16, 16)]
        wv = w_v[pl.ds(j * 16, 16)]
        plsc.addupdate_scatter(acc, [cv], wv)
        return carry

    lax.fori_loop(0, EPW // 16, edge_body, 0)
    pltpu.sync_copy(acc, out_hbm.at[wid])


@functools.partial(
    pl.kernel,
    out_type=jax.ShapeDtypeStruct((NC, NPAD, D_HID), jnp.float32),
    mesh=_mesh,
    compiler_params=_sc_params,
    scratch_types=[
        pltpu.VMEM((EPW,), jnp.int32),
        pltpu.VMEM((EPW,), jnp.int32),
        pltpu.VMEM((EPW,), jnp.float32),
        pltpu.VMEM((CHUNK, D_HID), jnp.float32),
        pltpu.VMEM((CHUNK, D_HID), jnp.float32),
        pltpu.VMEM((CHUNK, D_HID), jnp.float32),
        pltpu.VMEM((CHUNK, D_HID), jnp.float32),
        pltpu.SemaphoreType.DMA,
        pltpu.SemaphoreType.DMA,
        pltpu.SemaphoreType.DMA,
        pltpu.SemaphoreType.DMA,
        pltpu.SemaphoreType.DMA,
        pltpu.SemaphoreType.DMA,
        pltpu.SemaphoreType.DMA,
        pltpu.SemaphoreType.DMA,
        pltpu.VMEM_SHARED((NPAD, D_HID), jnp.float32),
    ],
)
def _prop_kernel(h_hbm, ei_hbm, w_hbm, zeros_hbm, out_hbm,
                 row_v, col_v, w_v, msgs0, msgs1, msgs2, msgs3,
                 gsem0, gsem1, gsem2, gsem3,
                 ssem0, ssem1, ssem2, ssem3, acc_sh):
    cid = lax.axis_index("c")
    sid = lax.axis_index("s")
    wid = sid * NC + cid
    r0 = sid * RPT
    pltpu.sync_copy(zeros_hbm.at[pl.ds(r0, RPT), :],
                    acc_sh.at[pl.ds(r0, RPT), :])
    pltpu.sync_copy(ei_hbm.at[0, pl.ds(wid * EPW, EPW)], row_v)
    pltpu.sync_copy(ei_hbm.at[1, pl.ds(wid * EPW, EPW)], col_v)
    pltpu.sync_copy(w_hbm.at[pl.ds(wid * EPW, EPW)], w_v)
    plsc.subcore_barrier()

    iota = lax.broadcasted_iota(jnp.int32, (16,), 0)
    zero16i = jnp.zeros((16,), jnp.int32)
    lane_idx = [jnp.full((16, 1), lane, jnp.int32) for lane in range(16)]
    _bcast_dnums = lax.GatherDimensionNumbers(
        offset_dims=(), collapsed_slice_dims=(0,), start_index_map=(0,)
    )

    def bcast(v, lane):
        return lax.gather(v, lane_idx[lane], _bcast_dnums, slice_sizes=(1,),
                          mode=lax.GatherScatterMode.PROMISE_IN_BOUNDS)

    def scale(msgs, j):
        jc = zero16i + j * CHUNK
        for e in range(CHUNK):
            wb = plsc.load_gather(w_v, [jc + e])
            for k in range(D_HID // 16):
                sl = pl.ds(16 * k, 16)
                msgs[e, sl] = msgs[e, sl] * wb

    bufs = (msgs0, msgs1, msgs2, msgs3)
    gsems = (gsem0, gsem1, gsem2, gsem3)
    ssems = (ssem0, ssem1, ssem2, ssem3)
    NB = 4

    def gather(j, b):
        idx = row_v.at[pl.ds(j * CHUNK, CHUNK)]
        return pltpu.async_copy(h_hbm.at[idx], bufs[b], gsems[b])

    def gwait(j, b):
        idx = row_v.at[pl.ds(j * CHUNK, CHUNK)]
        pltpu.make_async_copy(h_hbm.at[idx], bufs[b], gsems[b]).wait()

    def scatter(j, b):
        idx = col_v.at[pl.ds(j * CHUNK, CHUNK)]
        return pltpu.async_copy(bufs[b], acc_sh.at[idx], ssems[b], add=True)

    def swait(j, b):
        idx = col_v.at[pl.ds(j * CHUNK, CHUNK)]
        pltpu.make_async_copy(bufs[b], acc_sh.at[idx], ssems[b]).wait()

    gather(0, 0)
    gather(1, 1)

    NP = (NCH - 1) // NB

    def round_body(p, carry):
        for b in range(NB):
            j = p * NB + b
            jp = j + 2
            bp = (b + 2) % NB
            if b < 2:
                @pl.when(p >= 1)
                def _():
                    swait(jp - NB, bp)
                    gather(jp, bp)

                @pl.when(p == 0)
                def _():
                    gather(jp, bp)
            else:
                swait(jp - NB, bp)
                if b == 3:
                    @pl.when(p < NP - 1)
                    def _():
                        gather(jp, bp)
                else:
                    gather(jp, bp)
            gwait(j, b)
            scale(bufs[b], j)
            scatter(j, b)
        return carry

    lax.fori_loop(0, NP, round_body, 0)
    jt = NCH - 1
    gwait(jt, 0)
    scale(bufs[0], jt)
    scatter(jt, 0)
    swait(NCH - 3, 2)
    swait(NCH - 2, 3)
    swait(jt, 0)

    plsc.subcore_barrier()
    pltpu.sync_copy(acc_sh.at[pl.ds(r0, RPT), :],
                    out_hbm.at[cid, pl.ds(r0, RPT), :])


def _tc1_body(parts_ref, x_ref, w1_ref, h1s_ref, dis_ref):
    deg = jnp.sum(parts_ref[...], axis=0)[:N] + 1.0
    dis = lax.rsqrt(deg)
    h1 = jnp.dot(x_ref[...], w1_ref[...], preferred_element_type=jnp.float32)
    h1s_ref[...] = h1 * dis[:, None]
    dis_ref[...] = dis


def _tc2_body(acc_ref, h1s_ref, dis_ref, b1_ref, bnw_ref, bnb_ref, gs_ref):
    dis = dis_ref[...]
    agg = acc_ref[0, :N, :] + acc_ref[1, :N, :] + h1s_ref[...]
    h = agg * dis[:, None] + b1_ref[...][None, :]
    mean = jnp.mean(h, axis=0)
    var = jnp.mean((h - mean[None, :]) ** 2, axis=0)
    g = (h - mean[None, :]) * lax.rsqrt(var + BN_EPS)[None, :]
    g = g * bnw_ref[...][None, :] + bnb_ref[...][None, :]
    g = jnp.maximum(g, 0.0)
    gs_ref[...] = g * dis[:, None]


def _tc3_body(acc_ref, gs_ref, dis_ref, w2_ref, b2_ref, out_ref):
    agg = acc_ref[0, :N, :] + acc_ref[1, :N, :] + gs_ref[...]
    p2 = agg * dis_ref[...][:, None]
    out_ref[...] = (
        jnp.dot(p2, w2_ref[...], preferred_element_type=jnp.float32)
        + b2_ref[...][None, :]
    )


_tc1 = pl.pallas_call(
    _tc1_body,
    out_shape=(
        jax.ShapeDtypeStruct((N, D_HID), jnp.float32),
        jax.ShapeDtypeStruct((N,), jnp.float32),
    ),
)

_tc2 = pl.pallas_call(
    _tc2_body,
    out_shape=jax.ShapeDtypeStruct((N, D_HID), jnp.float32),
)

_tc3 = pl.pallas_call(
    _tc3_body,
    out_shape=jax.ShapeDtypeStruct((N, D_OUT), jnp.float32),
)


def kernel(x, edge_index, edge_weight, W1, b1, bn_weight, bn_bias, W2, b2):
    ei = edge_index.astype(jnp.int32)
    w = edge_weight.astype(jnp.float32)
    zeros2d = jnp.zeros((NPAD, D_HID), jnp.float32)

    deg_parts = _deg_kernel(ei, w)
    h1s, dis = _tc1(deg_parts, x, W1)
    acc1 = _prop_kernel(h1s, ei, w, zeros2d)
    gs = _tc2(acc1, h1s, dis, b1, bn_weight, bn_bias)
    acc2 = _prop_kernel(gs, ei, w, zeros2d)
    out = _tc3(acc2, gs, dis, W2, b2)
    return out


# speedup vs baseline: 2.0094x; 2.0094x over previous
import functools

import jax
import jax.numpy as jnp
from jax import lax
from jax.experimental import pallas as pl
from jax.experimental.pallas import tpu as pltpu
from jax.experimental.pallas import tpu_sc as plsc

N = 10000
E = 320000
D_IN = 128
D_HID = 64
D_OUT = 128
BN_EPS = 1e-5

NC = 2
NS = 16
NW = NC * NS
EPW = E // NW
CHUNK = 80
NCH = EPW // CHUNK
NPAD = 10240
RPT = NPAD // NS

_mesh = plsc.VectorSubcoreMesh(
    core_axis_name="c", subcore_axis_name="s", num_cores=NC, num_subcores=NS
)
_sc_params = pltpu.CompilerParams(
    needs_layout_passes=False, use_tc_tiling_on_sc=False
)


@functools.partial(
    pl.kernel,
    out_type=jax.ShapeDtypeStruct((NW, NPAD), jnp.float32),
    mesh=_mesh,
    compiler_params=_sc_params,
    scratch_types=[
        pltpu.VMEM((EPW,), jnp.int32),
        pltpu.VMEM((EPW,), jnp.float32),
        pltpu.VMEM((NPAD,), jnp.float32),
    ],
)
def _deg_kernel(ei_hbm, w_hbm, out_hbm, col_v, w_v, acc):
    cid = lax.axis_index("c")
    sid = lax.axis_index("s")
    wid = sid * NC + cid
    pltpu.sync_copy(ei_hbm.at[1, pl.ds(wid * EPW, EPW)], col_v)
    pltpu.sync_copy(w_hbm.at[pl.ds(wid * EPW, EPW)], w_v)

    zero16 = jnp.zeros((16,), jnp.float32)

    def zero_body(i, carry):
        acc[pl.ds(i * 16, 16)] = zero16
        return carry

    lax.fori_loop(0, NPAD // 16, zero_body, 0)

    def edge_body(j, carry):
        cv = col_v[pl.ds(j * 16, 16)]
        wv = w_v[pl.ds(j * 16, 16)]
        plsc.addupdate_scatter(acc, [cv], wv)
        return carry

    lax.fori_loop(0, EPW // 16, edge_body, 0)
    pltpu.sync_copy(acc, out_hbm.at[wid])


@functools.partial(
    pl.kernel,
    out_type=jax.ShapeDtypeStruct((NC, NPAD, D_HID), jnp.float32),
    mesh=_mesh,
    compiler_params=_sc_params,
    scratch_types=[
        pltpu.VMEM((EPW,), jnp.int32),
        pltpu.VMEM((EPW,), jnp.int32),
        pltpu.VMEM((EPW,), jnp.float32),
        pltpu.VMEM((CHUNK, D_HID), jnp.float32),
        pltpu.VMEM((CHUNK, D_HID), jnp.float32),
        pltpu.VMEM((CHUNK, D_HID), jnp.float32),
        pltpu.VMEM((CHUNK, D_HID), jnp.float32),
        pltpu.VMEM((CHUNK, D_HID), jnp.float32),
        pltpu.VMEM((CHUNK, D_HID), jnp.float32),
        pltpu.VMEM((CHUNK, D_HID), jnp.float32),
        pltpu.VMEM((CHUNK, D_HID), jnp.float32),
        pltpu.SemaphoreType.DMA,
        pltpu.SemaphoreType.DMA,
        pltpu.SemaphoreType.DMA,
        pltpu.SemaphoreType.DMA,
        pltpu.SemaphoreType.DMA,
        pltpu.SemaphoreType.DMA,
        pltpu.SemaphoreType.DMA,
        pltpu.SemaphoreType.DMA,
        pltpu.VMEM_SHARED((NPAD, D_HID), jnp.float32),
    ],
)
def _prop_kernel(h_hbm, ei_hbm, w_hbm, zeros_hbm, out_hbm,
                 row_v, col_v, w_v, msgs0, msgs1, msgs2, msgs3,
                 obuf0, obuf1, obuf2, obuf3,
                 gsem0, gsem1, gsem2, gsem3,
                 ssem0, ssem1, ssem2, ssem3, acc_sh):
    cid = lax.axis_index("c")
    sid = lax.axis_index("s")
    wid = sid * NC + cid
    r0 = sid * RPT
    pltpu.sync_copy(zeros_hbm.at[pl.ds(r0, RPT), :],
                    acc_sh.at[pl.ds(r0, RPT), :])
    pltpu.sync_copy(ei_hbm.at[0, pl.ds(wid * EPW, EPW)], row_v)
    pltpu.sync_copy(ei_hbm.at[1, pl.ds(wid * EPW, EPW)], col_v)
    pltpu.sync_copy(w_hbm.at[pl.ds(wid * EPW, EPW)], w_v)
    plsc.subcore_barrier()

    iota = lax.broadcasted_iota(jnp.int32, (16,), 0)
    zero16i = jnp.zeros((16,), jnp.int32)
    lane_idx = [jnp.full((16, 1), lane, jnp.int32) for lane in range(16)]
    _bcast_dnums = lax.GatherDimensionNumbers(
        offset_dims=(), collapsed_slice_dims=(0,), start_index_map=(0,)
    )

    def bcast(v, lane):
        return lax.gather(v, lane_idx[lane], _bcast_dnums, slice_sizes=(1,),
                          mode=lax.GatherScatterMode.PROMISE_IN_BOUNDS)

    def scale(src, dst, j):
        for m in range(CHUNK // 16):
            wv = w_v[pl.ds(j * CHUNK + 16 * m, 16)]
            for lane in range(16):
                e = m * 16 + lane
                wb = bcast(wv, lane)
                for k in range(D_HID // 16):
                    sl = pl.ds(16 * k, 16)
                    dst[e, sl] = src[e, sl] * wb

    bufs = (msgs0, msgs1, msgs2, msgs3)
    obufs = (obuf0, obuf1, obuf2, obuf3)
    gsems = (gsem0, gsem1, gsem2, gsem3)
    ssems = (ssem0, ssem1, ssem2, ssem3)
    NB = 4

    def gather(j, b):
        idx = row_v.at[pl.ds(j * CHUNK, CHUNK)]
        return pltpu.async_copy(h_hbm.at[idx], bufs[b], gsems[b])

    def gwait(j, b):
        idx = row_v.at[pl.ds(j * CHUNK, CHUNK)]
        pltpu.make_async_copy(h_hbm.at[idx], bufs[b], gsems[b]).wait()

    def scatter(j, b):
        idx = col_v.at[pl.ds(j * CHUNK, CHUNK)]
        return pltpu.async_copy(obufs[b], acc_sh.at[idx], ssems[b], add=True)

    def swait(j, b):
        idx = col_v.at[pl.ds(j * CHUNK, CHUNK)]
        pltpu.make_async_copy(obufs[b], acc_sh.at[idx], ssems[b]).wait()

    gather(0, 0)
    gather(1, 1)

    NP = (NCH - 1) // NB

    def round_body(p, carry):
        for b in range(NB):
            j = p * NB + b
            jp = j + 2
            bp = (b + 2) % NB
            if b < 2:
                @pl.when(p >= 1)
                def _():
                    swait(jp - NB, bp)
                    gather(jp, bp)

                @pl.when(p == 0)
                def _():
                    gather(jp, bp)
            else:
                swait(jp - NB, bp)
                if b == 3:
                    @pl.when(p < NP - 1)
                    def _():
                        gather(jp, bp)
                else:
                    gather(jp, bp)
            gwait(j, b)
            scale(bufs[b], obufs[b], j)
            scatter(j, b)
        return carry

    lax.fori_loop(0, NP, round_body, 0)
    jt = NCH - 1
    gwait(jt, 0)
    scale(bufs[0], obufs[0], jt)
    scatter(jt, 0)
    swait(NCH - 3, 2)
    swait(NCH - 2, 3)
    swait(jt, 0)

    plsc.subcore_barrier()
    pltpu.sync_copy(acc_sh.at[pl.ds(r0, RPT), :],
                    out_hbm.at[cid, pl.ds(r0, RPT), :])


def _tc1_body(parts_ref, x_ref, w1_ref, h1s_ref, dis_ref):
    deg = jnp.sum(parts_ref[...], axis=0)[:N] + 1.0
    dis = lax.rsqrt(deg)
    h1 = jnp.dot(x_ref[...], w1_ref[...], preferred_element_type=jnp.float32)
    h1s_ref[...] = h1 * dis[:, None]
    dis_ref[...] = dis


def _tc2_body(acc_ref, h1s_ref, dis_ref, b1_ref, bnw_ref, bnb_ref, gs_ref):
    dis = dis_ref[...]
    agg = acc_ref[0, :N, :] + acc_ref[1, :N, :] + h1s_ref[...]
    h = agg * dis[:, None] + b1_ref[...][None, :]
    mean = jnp.mean(h, axis=0)
    var = jnp.mean((h - mean[None, :]) ** 2, axis=0)
    g = (h - mean[None, :]) * lax.rsqrt(var + BN_EPS)[None, :]
    g = g * bnw_ref[...][None, :] + bnb_ref[...][None, :]
    g = jnp.maximum(g, 0.0)
    gs_ref[...] = g * dis[:, None]


def _tc3_body(acc_ref, gs_ref, dis_ref, w2_ref, b2_ref, out_ref):
    agg = acc_ref[0, :N, :] + acc_ref[1, :N, :] + gs_ref[...]
    p2 = agg * dis_ref[...][:, None]
    out_ref[...] = (
        jnp.dot(p2, w2_ref[...], preferred_element_type=jnp.float32)
        + b2_ref[...][None, :]
    )


_tc1 = pl.pallas_call(
    _tc1_body,
    out_shape=(
        jax.ShapeDtypeStruct((N, D_HID), jnp.float32),
        jax.ShapeDtypeStruct((N,), jnp.float32),
    ),
)

_tc2 = pl.pallas_call(
    _tc2_body,
    out_shape=jax.ShapeDtypeStruct((N, D_HID), jnp.float32),
)

_tc3 = pl.pallas_call(
    _tc3_body,
    out_shape=jax.ShapeDtypeStruct((N, D_OUT), jnp.float32),
)


def kernel(x, edge_index, edge_weight, W1, b1, bn_weight, bn_bias, W2, b2):
    ei = edge_index.astype(jnp.int32)
    w = edge_weight.astype(jnp.float32)
    zeros2d = jnp.zeros((NPAD, D_HID), jnp.float32)

    deg_parts = _deg_kernel(ei, w)
    h1s, dis = _tc1(deg_parts, x, W1)
    acc1 = _prop_kernel(h1s, ei, w, zeros2d)
    gs = _tc2(acc1, h1s, dis, b1, bn_weight, bn_bias)
    acc2 = _prop_kernel(gs, ei, w, zeros2d)
    out = _tc3(acc2, gs, dis, W2, b2)
    return out
